# baseline (speedup 1.0000x reference)
import jax
import jax.numpy as jnp
from jax import lax
from jax.experimental import pallas as pl
from jax.experimental.pallas import tpu as pltpu
from jax.experimental.pallas import tpu_sc as plsc

NUM_ACTIONS = 16
D = 128
NUM_ARGS = 3
L_TOTAL = 819200

NC = 2
NS = 16
NW = NC * NS
CHUNK = 256
GATHER = 128


def _build_combined(table):
    def body(t_ref, out_ref):
        t = t_ref[...]
        tb = jnp.broadcast_to(t[None], (NUM_ARGS, NUM_ACTIONS, D - 1))
        g = lax.broadcasted_iota(jnp.int32, (NUM_ARGS, NUM_ACTIONS, 1), 0).astype(jnp.float32)
        comb = jnp.concatenate([tb, g], axis=2)
        comb = comb.reshape(NUM_ACTIONS * NUM_ARGS, D)
        rep = jnp.broadcast_to(comb[None], (NW, NUM_ACTIONS * NUM_ARGS, D))
        out_ref[...] = rep.reshape(NW * NUM_ACTIONS * NUM_ARGS, D)

    return pl.pallas_call(
        body,
        out_shape=jax.ShapeDtypeStruct((NW * NUM_ACTIONS * NUM_ARGS, D), jnp.float32),
    )(table)


def _sc_lookup_body(comb_hbm, act_hbm, arg_hbm, out_hbm,
                    tbl_v, a0, a1, g0, g1, rows0, rows1,
                    sem_i0, sem_i1, sem_o0, sem_o1):
    rows_per_w = L_TOTAL // NW
    n_chunks = rows_per_w // CHUNK
    wid = lax.axis_index("s") * NC + lax.axis_index("c")
    w_base = wid * rows_per_w
    rows = [rows0, rows1]
    a_v = [a0, a1]
    g_v = [g0, g1]
    sem_i = [sem_i0, sem_i1]
    sem_o = [sem_o0, sem_o1]
    tbl_base = wid * (NUM_ACTIONS * NUM_ARGS)

    pltpu.sync_copy(comb_hbm.at[pl.ds(tbl_base, NUM_ACTIONS * NUM_ARGS)], tbl_v)

    def fire_inputs(c, b):
        base = w_base + c * CHUNK
        pltpu.async_copy(act_hbm.at[pl.ds(base, CHUNK)], a_v[b], sem_i[b])
        pltpu.async_copy(arg_hbm.at[pl.ds(base, CHUNK)], g_v[b], sem_i[b])

    def drain_inputs(b):
        pltpu.make_async_copy(
            act_hbm.at[pl.ds(w_base, CHUNK)], a_v[b], sem_i[b]).wait()
        pltpu.make_async_copy(
            arg_hbm.at[pl.ds(w_base, CHUNK)], g_v[b], sem_i[b]).wait()

    def drain_out(b):
        pltpu.make_async_copy(
            rows[b], out_hbm.at[pl.ds(w_base, CHUNK)], sem_o[b]).wait()

    def compute_chunk(b):
        lane = lax.iota(jnp.int32, 16)

        def block(ib, carry):
            riota = ib * 16 + lane
            av = plsc.load_gather(a_v[b], [riota])
            gv = plsc.load_gather(g_v[b], [riota])
            rvec = gv * NUM_ACTIONS + av
            for d in range(D):
                dvec = jnp.full((16,), d, jnp.int32)
                colv = plsc.load_gather(tbl_v, [rvec, dvec])
                plsc.store_scatter(rows[b], [riota, dvec], colv)
            return carry

        lax.fori_loop(0, CHUNK // 16, block, 0, unroll=False)

    def step(c, b, drain_prev_out, prefetch_c):
        nb = 1 - b
        fire_inputs(prefetch_c, nb)
        if drain_prev_out:
            drain_out(b)
        drain_inputs(b)
        compute_chunk(b)
        pltpu.async_copy(
            rows[b], out_hbm.at[pl.ds(w_base + c * CHUNK, CHUNK)], sem_o[b])

    fire_inputs(0, 0)
    step(0, 0, False, 1)
    step(1, 1, False, 2)

    def pair(k, carry):
        c = 2 + 2 * k
        step(c, 0, True, c + 1)
        step(c + 1, 1, True, c + 2 - 2 * (k == (n_chunks - 4) // 2))
        return carry

    lax.fori_loop(0, (n_chunks - 2) // 2, pair, 0, unroll=False)
    drain_inputs(0)
    drain_out(0)
    drain_out(1)


@jax.jit
def kernel(actions, arguments, table):
    comb = _build_combined(table.astype(jnp.float32))
    act = actions.astype(jnp.int32)
    arg = arguments.astype(jnp.int32)

    mesh = plsc.VectorSubcoreMesh(core_axis_name="c", subcore_axis_name="s")
    lookup = pl.kernel(
        _sc_lookup_body,
        out_type=jax.ShapeDtypeStruct((L_TOTAL, D), jnp.float32),
        mesh=mesh,
        compiler_params=pltpu.CompilerParams(needs_layout_passes=False),
        scratch_types=[
            pltpu.VMEM((NUM_ACTIONS * NUM_ARGS, D), jnp.float32),
            pltpu.VMEM((CHUNK,), jnp.int32),
            pltpu.VMEM((CHUNK,), jnp.int32),
            pltpu.VMEM((CHUNK,), jnp.int32),
            pltpu.VMEM((CHUNK,), jnp.int32),
            pltpu.VMEM((CHUNK, D), jnp.float32),
            pltpu.VMEM((CHUNK, D), jnp.float32),
            pltpu.SemaphoreType.DMA,
            pltpu.SemaphoreType.DMA,
            pltpu.SemaphoreType.DMA,
            pltpu.SemaphoreType.DMA,
        ],
    )
    return lookup(comb, act, arg)


# speedup vs baseline: 14.0130x; 14.0130x over previous
import jax
import jax.numpy as jnp
from jax import lax
from jax.experimental import pallas as pl
from jax.experimental.pallas import tpu as pltpu
from jax.experimental.pallas import tpu_sc as plsc

NUM_ACTIONS = 16
D = 128
NUM_ARGS = 3
L_TOTAL = 819200

NC = 2
NS = 16
NW = NC * NS
CHUNK = 256
GATHER = 128


def _build_combined(table):
    def body(t_ref, out_ref):
        t = t_ref[...]
        tb = jnp.broadcast_to(t[None], (NUM_ARGS, NUM_ACTIONS, D - 1))
        g = lax.broadcasted_iota(jnp.int32, (NUM_ARGS, NUM_ACTIONS, 1), 0).astype(jnp.float32)
        comb = jnp.concatenate([tb, g], axis=2)
        comb = comb.reshape(NUM_ACTIONS * NUM_ARGS, D)
        rep = jnp.broadcast_to(comb[None], (NW, NUM_ACTIONS * NUM_ARGS, D))
        out_ref[...] = rep.reshape(NW * NUM_ACTIONS * NUM_ARGS, D)

    return pl.pallas_call(
        body,
        out_shape=jax.ShapeDtypeStruct((NW * NUM_ACTIONS * NUM_ARGS, D), jnp.float32),
    )(table)


def _sc_lookup_body(comb_hbm, act_hbm, arg_hbm, out_hbm,
                    tbl_sh, a_v, g_v, idx0, idx1, rows0, rows1,
                    sem_g0, sem_g1, sem_o0, sem_o1):
    rows_per_w = L_TOTAL // NW
    n_chunks = rows_per_w // CHUNK
    sid = lax.axis_index("s")
    wid = sid * NC + lax.axis_index("c")
    w_base = wid * rows_per_w
    rows = [rows0, rows1]
    idx = [idx0, idx1]
    sem_g = [sem_g0, sem_g1]
    sem_o = [sem_o0, sem_o1]

    @pl.when(sid == 0)
    def _stage():
        pltpu.sync_copy(
            comb_hbm.at[pl.ds(0, NUM_ACTIONS * NUM_ARGS)], tbl_sh)

    plsc.subcore_barrier()

    def fire_gathers(c, b):
        base = w_base + c * CHUNK
        pltpu.sync_copy(act_hbm.at[pl.ds(base, CHUNK)], a_v)
        pltpu.sync_copy(arg_hbm.at[pl.ds(base, CHUNK)], g_v)
        for i in range(CHUNK // 16):
            a = a_v[pl.ds(i * 16, 16)]
            g = g_v[pl.ds(i * 16, 16)]
            idx[b][i // 8, pl.ds((i % 8) * 16, 16)] = g * NUM_ACTIONS + a
        for j in range(CHUNK // GATHER):
            pltpu.async_copy(
                tbl_sh.at[idx[b].at[j]],
                rows[b].at[pl.ds(j * GATHER, GATHER)],
                sem_g[b],
            )

    def drain_gathers(b):
        for j in range(CHUNK // GATHER):
            pltpu.make_async_copy(
                tbl_sh.at[idx[b].at[j]],
                rows[b].at[pl.ds(j * GATHER, GATHER)],
                sem_g[b],
            ).wait()

    def drain_out(b):
        pltpu.make_async_copy(
            rows[b], out_hbm.at[pl.ds(w_base, CHUNK)], sem_o[b]).wait()

    def step(c, b, drain_prev_out, prefetch):
        nb = 1 - b
        if prefetch:
            if drain_prev_out:
                drain_out(nb)
            fire_gathers(c + 1, nb)
        drain_gathers(b)
        pltpu.async_copy(
            rows[b], out_hbm.at[pl.ds(w_base + c * CHUNK, CHUNK)], sem_o[b])

    fire_gathers(0, 0)
    step(0, 0, False, True)

    def pair(k, carry):
        c = 1 + 2 * k
        step(c, 1, True, True)
        step(c + 1, 0, True, True)
        return carry

    lax.fori_loop(0, (n_chunks - 2) // 2, pair, 0, unroll=False)
    step(n_chunks - 1, 1, False, False)
    drain_out(0)
    drain_out(1)


@jax.jit
def kernel(actions, arguments, table):
    comb = _build_combined(table.astype(jnp.float32))
    act = actions.astype(jnp.int32)
    arg = arguments.astype(jnp.int32)

    mesh = plsc.VectorSubcoreMesh(core_axis_name="c", subcore_axis_name="s")
    lookup = pl.kernel(
        _sc_lookup_body,
        out_type=jax.ShapeDtypeStruct((L_TOTAL, D), jnp.float32),
        mesh=mesh,
        scratch_types=[
            pltpu.VMEM_SHARED((NUM_ACTIONS * NUM_ARGS, D), jnp.float32),
            pltpu.VMEM((CHUNK,), jnp.int32),
            pltpu.VMEM((CHUNK,), jnp.int32),
            pltpu.VMEM((CHUNK // GATHER, GATHER), jnp.int32),
            pltpu.VMEM((CHUNK // GATHER, GATHER), jnp.int32),
            pltpu.VMEM((CHUNK, D), jnp.float32),
            pltpu.VMEM((CHUNK, D), jnp.float32),
            pltpu.SemaphoreType.DMA,
            pltpu.SemaphoreType.DMA,
            pltpu.SemaphoreType.DMA,
            pltpu.SemaphoreType.DMA,
        ],
    )
    return lookup(comb, act, arg)


# speedup vs baseline: 19.8419x; 1.4160x over previous
import jax
import jax.numpy as jnp
from jax import lax
from jax.experimental import pallas as pl
from jax.experimental.pallas import tpu as pltpu
from jax.experimental.pallas import tpu_sc as plsc

NUM_ACTIONS = 16
D = 128
NUM_ARGS = 3
L_TOTAL = 819200

NC = 2
NS = 16
NW = NC * NS
CHUNK = 256
GATHER = 128


def _build_combined(table):
    def body(t_ref, out_ref):
        t = t_ref[...]
        tb = jnp.broadcast_to(t[None], (NUM_ARGS, NUM_ACTIONS, D - 1))
        g = lax.broadcasted_iota(jnp.int32, (NUM_ARGS, NUM_ACTIONS, 1), 0).astype(jnp.float32)
        comb = jnp.concatenate([tb, g], axis=2)
        comb = comb.reshape(NUM_ACTIONS * NUM_ARGS, D)
        rep = jnp.broadcast_to(comb[None], (NW, NUM_ACTIONS * NUM_ARGS, D))
        out_ref[...] = rep.reshape(NW * NUM_ACTIONS * NUM_ARGS, D)

    return pl.pallas_call(
        body,
        out_shape=jax.ShapeDtypeStruct((NW * NUM_ACTIONS * NUM_ARGS, D), jnp.float32),
    )(table)


def _sc_lookup_body(comb_hbm, act_hbm, arg_hbm, out_hbm,
                    tbl_sh, a_v, g_v, idx0, idx1, idx2, rows0, rows1, rows2,
                    sem_g0, sem_g1, sem_g2, sem_o0, sem_o1, sem_o2):
    rows_per_w = L_TOTAL // NW
    n_chunks = rows_per_w // CHUNK
    sid = lax.axis_index("s")
    wid = sid * NC + lax.axis_index("c")
    w_base = wid * rows_per_w
    rows = [rows0, rows1, rows2]
    idx = [idx0, idx1, idx2]
    sem_g = [sem_g0, sem_g1, sem_g2]
    sem_o = [sem_o0, sem_o1, sem_o2]

    @pl.when(sid == 0)
    def _stage():
        pltpu.sync_copy(
            comb_hbm.at[pl.ds(0, NUM_ACTIONS * NUM_ARGS)], tbl_sh)

    plsc.subcore_barrier()

    def fire_gathers(c, b):
        base = w_base + c * CHUNK
        pltpu.sync_copy(act_hbm.at[pl.ds(base, CHUNK)], a_v)
        pltpu.sync_copy(arg_hbm.at[pl.ds(base, CHUNK)], g_v)
        for i in range(CHUNK // 16):
            a = a_v[pl.ds(i * 16, 16)]
            g = g_v[pl.ds(i * 16, 16)]
            idx[b][i // 8, pl.ds((i % 8) * 16, 16)] = g * NUM_ACTIONS + a
        for j in range(CHUNK // GATHER):
            pltpu.async_copy(
                tbl_sh.at[idx[b].at[j]],
                rows[b].at[pl.ds(j * GATHER, GATHER)],
                sem_g[b],
            )

    def drain_gathers(b):
        for j in range(CHUNK // GATHER):
            pltpu.make_async_copy(
                tbl_sh.at[idx[b].at[j]],
                rows[b].at[pl.ds(j * GATHER, GATHER)],
                sem_g[b],
            ).wait()

    def drain_out(b):
        pltpu.make_async_copy(
            rows[b], out_hbm.at[pl.ds(w_base, CHUNK)], sem_o[b]).wait()

    def step(c, b, drain_prev_out, prefetch):
        nb = (b + 1) % 3
        if drain_prev_out:
            drain_out(nb)
        if prefetch:
            fire_gathers(c + 1, nb)
        drain_gathers(b)
        pltpu.async_copy(
            rows[b], out_hbm.at[pl.ds(w_base + c * CHUNK, CHUNK)], sem_o[b])

    fire_gathers(0, 0)
    step(0, 0, False, True)
    step(1, 1, False, True)
    step(2, 2, True, True)

    def triple(k, carry):
        c = 3 + 3 * k
        step(c, 0, True, True)
        step(c + 1, 1, True, True)
        step(c + 2, 2, True, True)
        return carry

    lax.fori_loop(0, (n_chunks - 4) // 3, triple, 0, unroll=False)
    step(n_chunks - 1, 0, True, False)
    drain_out(2)
    drain_out(0)


@jax.jit
def kernel(actions, arguments, table):
    comb = _build_combined(table.astype(jnp.float32))
    act = actions.astype(jnp.int32)
    arg = arguments.astype(jnp.int32)

    mesh = plsc.VectorSubcoreMesh(core_axis_name="c", subcore_axis_name="s")
    lookup = pl.kernel(
        _sc_lookup_body,
        out_type=jax.ShapeDtypeStruct((L_TOTAL, D), jnp.float32),
        mesh=mesh,
        scratch_types=[
            pltpu.VMEM_SHARED((NUM_ACTIONS * NUM_ARGS, D), jnp.float32),
            pltpu.VMEM((CHUNK,), jnp.int32),
            pltpu.VMEM((CHUNK,), jnp.int32),
            pltpu.VMEM((CHUNK // GATHER, GATHER), jnp.int32),
            pltpu.VMEM((CHUNK // GATHER, GATHER), jnp.int32),
            pltpu.VMEM((CHUNK // GATHER, GATHER), jnp.int32),
            pltpu.VMEM((CHUNK, D), jnp.float32),
            pltpu.VMEM((CHUNK, D), jnp.float32),
            pltpu.VMEM((CHUNK, D), jnp.float32),
            pltpu.SemaphoreType.DMA,
            pltpu.SemaphoreType.DMA,
            pltpu.SemaphoreType.DMA,
            pltpu.SemaphoreType.DMA,
            pltpu.SemaphoreType.DMA,
            pltpu.SemaphoreType.DMA,
        ],
    )
    return lookup(comb, act, arg)
